# Initial kernel scaffold; baseline (speedup 1.0000x reference)
import jax
import jax.numpy as jnp
from jax.experimental import pallas as pl


def kernel(schemas, logits):
    raise NotImplementedError("write your pallas kernel here")


# speedup vs baseline: 18.9282x; 18.9282x over previous
import jax
import jax.numpy as jnp
import numpy as np
from jax import lax
from jax.experimental import pallas as pl
from jax.experimental.pallas import tpu as pltpu
from jax.experimental.pallas import tpu_sc as plsc

B, D, L = 16, 64, 4096
HEAD = 64
NC, NS = 2, 16
ZPW = D // 2
CH = 8
LANES = 16

_T2 = np.asarray(np.arange(HEAD)[None, :] < np.arange(HEAD)[:, None], dtype=np.int8)
_ZROW_F = np.zeros((L,), np.float32)
_ZROW_B = np.zeros((L,), np.int8)


def _sc_body(schemas_hbm, logits_hbm, t2_hbm, zf_hbm, zb_hbm,
             outl_hbm, outm_hbm,
             logits_v, schemas_v, starts_v, mheads_v, rowbuf, maskbuf, sem):
    cid = lax.axis_index("c")
    sid = lax.axis_index("s")
    wid = sid * NC + cid
    b = wid // 2
    half = wid % 2
    d0 = half * ZPW

    pltpu.sync_copy(schemas_hbm.at[b], schemas_v)
    pltpu.sync_copy(logits_hbm.at[b], logits_v)
    pltpu.async_copy(t2_hbm.at[schemas_v.at[pl.ds(d0, ZPW)]], mheads_v, sem).wait()

    carry = jnp.int32(0)
    for ci in range(D // LANES):
        seg = schemas_v[pl.ds(ci * LANES, LANES)]
        inc = plsc.cumsum(seg)
        starts_v[pl.ds(ci * LANES, LANES)] = inc - seg + carry
        carry = carry + jnp.sum(seg)

    for r in range(CH):
        pltpu.sync_copy(zf_hbm, rowbuf.at[r])
        pltpu.sync_copy(zb_hbm, maskbuf.at[r])

    iota = lax.iota(jnp.int32, LANES)
    for k in range(ZPW // CH):
        for r in range(CH):
            z = k * CH + r
            dg = d0 + z
            idxv = jnp.full((LANES,), dg, jnp.int32)
            nd = plsc.load_gather(schemas_v, [idxv])
            sd = plsc.load_gather(starts_v, [idxv])
            for c4 in range(HEAD // LANES):
                j = iota + (c4 * LANES)
                m = j < nd
                gi = jnp.where(m, sd + j, 0)
                vals = plsc.load_gather(logits_v, [gi])
                rowbuf[r, pl.ds(c4 * LANES, LANES)] = jnp.where(m, vals, 0.0)
            maskbuf[r, pl.ds(0, HEAD)] = mheads_v[z, :]
        rows0 = d0 + k * CH
        pltpu.sync_copy(rowbuf, outl_hbm.at[b, pl.ds(rows0, CH)])
        pltpu.sync_copy(maskbuf, outm_hbm.at[b, pl.ds(rows0, CH)])


def kernel(schemas, logits):
    schemas = schemas.astype(jnp.int32)
    logits = logits.astype(jnp.float32)
    mesh = plsc.VectorSubcoreMesh(core_axis_name="c", subcore_axis_name="s",
                                  num_cores=NC, num_subcores=NS)
    run = pl.kernel(
        _sc_body,
        out_type=[jax.ShapeDtypeStruct((B, D, L), jnp.float32),
                  jax.ShapeDtypeStruct((B, D, L), jnp.int8)],
        mesh=mesh,
        compiler_params=pltpu.CompilerParams(use_tc_tiling_on_sc=False,
                                             needs_layout_passes=False),
        scratch_types=[
            pltpu.VMEM((L,), jnp.float32),
            pltpu.VMEM((D,), jnp.int32),
            pltpu.VMEM((D,), jnp.int32),
            pltpu.VMEM((ZPW, HEAD), jnp.int8),
            pltpu.VMEM((CH, L), jnp.float32),
            pltpu.VMEM((CH, L), jnp.int8),
            pltpu.SemaphoreType.DMA,
        ],
    )
    out_l, out_m = run(schemas, logits, jnp.asarray(_T2),
                       jnp.asarray(_ZROW_F), jnp.asarray(_ZROW_B))
    return out_l, out_m.astype(jnp.bool_)


# speedup vs baseline: 42.6838x; 2.2550x over previous
import jax
import jax.numpy as jnp
import numpy as np
from jax import lax
from jax.experimental import pallas as pl
from jax.experimental.pallas import tpu as pltpu
from jax.experimental.pallas import tpu_sc as plsc

B, D, L = 16, 64, 4096
HEAD = 64
HPAD = 128
NC, NS = 2, 16
ZPW = D // 2
LANES = 16

_ZHEAD = np.zeros((NC * NS * ZPW * HPAD // (NC * NS),), np.float32)


def _sc_heads_body(schemas_hbm, logits_hbm, zf_hbm, heads_hbm,
                   logits_v, schemas_v, starts_v, hbuf, sem):
    cid = lax.axis_index("c")
    sid = lax.axis_index("s")
    wid = sid * NC + cid
    b = wid // 2
    d0 = (wid % 2) * ZPW

    pltpu.sync_copy(schemas_hbm.at[b], schemas_v)
    pltpu.sync_copy(logits_hbm.at[b], logits_v)
    pltpu.sync_copy(zf_hbm, hbuf)

    carry = jnp.int32(0)
    for ci in range(D // LANES):
        seg = schemas_v[pl.ds(ci * LANES, LANES)]
        inc = plsc.cumsum(seg)
        starts_v[pl.ds(ci * LANES, LANES)] = inc - seg + carry
        carry = carry + jnp.sum(seg)

    iota = lax.iota(jnp.int32, LANES)
    for z in range(ZPW):
        idxv = jnp.full((LANES,), d0 + z, jnp.int32)
        nd = plsc.load_gather(schemas_v, [idxv])
        sd = plsc.load_gather(starts_v, [idxv])
        for c4 in range(HEAD // LANES):
            j = iota + (c4 * LANES)
            m = j < nd
            gi = jnp.where(m, sd + j, 0)
            vals = plsc.load_gather(logits_v, [gi])
            hbuf[pl.ds(z * HPAD + c4 * LANES, LANES)] = jnp.where(m, vals, 0.0)
    pltpu.sync_copy(hbuf, heads_hbm.at[pl.ds(wid * (ZPW * HPAD), ZPW * HPAD)])


def _tc_assemble_body(heads_ref, schemas_ref, outl_ref, outm_ref):
    n = schemas_ref[0, 0, :]
    col = lax.broadcasted_iota(jnp.int32, (D, L), 1)
    outm_ref[0] = col < n[:, None]
    outl_ref[0, :, pl.ds(0, HPAD)] = heads_ref[0]
    outl_ref[0, :, pl.ds(HPAD, L - HPAD)] = jnp.zeros((D, L - HPAD), jnp.float32)


def kernel(schemas, logits):
    schemas = schemas.astype(jnp.int32)
    logits = logits.astype(jnp.float32)

    mesh = plsc.VectorSubcoreMesh(core_axis_name="c", subcore_axis_name="s",
                                  num_cores=NC, num_subcores=NS)
    sc_heads = pl.kernel(
        _sc_heads_body,
        out_type=jax.ShapeDtypeStruct((B * D * HPAD,), jnp.float32),
        mesh=mesh,
        compiler_params=pltpu.CompilerParams(use_tc_tiling_on_sc=False,
                                             needs_layout_passes=False),
        scratch_types=[
            pltpu.VMEM((L,), jnp.float32),
            pltpu.VMEM((D,), jnp.int32),
            pltpu.VMEM((D,), jnp.int32),
            pltpu.VMEM((ZPW * HPAD,), jnp.float32),
            pltpu.SemaphoreType.DMA,
        ],
    )
    heads = sc_heads(schemas, logits, jnp.asarray(_ZHEAD)).reshape(B, D, HPAD)

    out_l, out_m = pl.pallas_call(
        _tc_assemble_body,
        grid=(B,),
        in_specs=[
            pl.BlockSpec((1, D, HPAD), lambda i: (i, 0, 0)),
            pl.BlockSpec((1, 1, D), lambda i: (i, 0, 0)),
        ],
        out_specs=[
            pl.BlockSpec((1, D, L), lambda i: (i, 0, 0)),
            pl.BlockSpec((1, D, L), lambda i: (i, 0, 0)),
        ],
        out_shape=[jax.ShapeDtypeStruct((B, D, L), jnp.float32),
                   jax.ShapeDtypeStruct((B, D, L), jnp.bool_)],
    )(heads, schemas.reshape(B, 1, D))
    return out_l, out_m


# speedup vs baseline: 46.7161x; 1.0945x over previous
import jax
import jax.numpy as jnp
import numpy as np
from jax import lax
from jax.experimental import pallas as pl
from jax.experimental.pallas import tpu as pltpu
from jax.experimental.pallas import tpu_sc as plsc

B, D, L = 16, 64, 4096
HEAD = 64
HPAD = 128
NC, NS = 2, 16
ZPW = D // 2
LANES = 16

_ZHEAD = np.zeros((NC * NS * ZPW * HPAD // (NC * NS),), np.float32)


def _sc_heads_body(schemas_hbm, logits_hbm, zf_hbm, heads_hbm,
                   logits_v, schemas_v, starts_v, hbuf, sem):
    cid = lax.axis_index("c")
    sid = lax.axis_index("s")
    wid = sid * NC + cid
    b = wid // 2
    d0 = (wid % 2) * ZPW

    pltpu.sync_copy(schemas_hbm.at[b], schemas_v)
    pltpu.sync_copy(logits_hbm.at[b], logits_v)
    pltpu.sync_copy(zf_hbm, hbuf)

    carry = jnp.int32(0)
    for ci in range(D // LANES):
        seg = schemas_v[pl.ds(ci * LANES, LANES)]
        inc = plsc.cumsum(seg)
        starts_v[pl.ds(ci * LANES, LANES)] = inc - seg + carry
        carry = carry + jnp.sum(seg)

    iota = lax.iota(jnp.int32, LANES)
    for z in range(ZPW):
        idxv = jnp.full((LANES,), d0 + z, jnp.int32)
        nd = plsc.load_gather(schemas_v, [idxv])
        sd = plsc.load_gather(starts_v, [idxv])
        for c4 in range(HEAD // LANES):
            j = iota + (c4 * LANES)
            m = j < nd
            gi = jnp.where(m, sd + j, 0)
            vals = plsc.load_gather(logits_v, [gi])
            hbuf[pl.ds(z * HPAD + c4 * LANES, LANES)] = jnp.where(m, vals, 0.0)
    pltpu.sync_copy(hbuf, heads_hbm.at[pl.ds(wid * (ZPW * HPAD), ZPW * HPAD)])


def _tc_mask_body(schemas_ref, outm_ref):
    n = schemas_ref[0, 0, :]
    col = lax.broadcasted_iota(jnp.int32, (D, L), 1)
    outm_ref[0] = (col < n[:, None]).astype(jnp.int8)


def _tc_assemble_body(heads_ref, outl_ref):
    outl_ref[0, :, pl.ds(0, HPAD)] = heads_ref[0]
    outl_ref[0, :, pl.ds(HPAD, L - HPAD)] = jnp.zeros((D, L - HPAD), jnp.float32)


def kernel(schemas, logits):
    schemas = schemas.astype(jnp.int32)
    logits = logits.astype(jnp.float32)

    mesh = plsc.VectorSubcoreMesh(core_axis_name="c", subcore_axis_name="s",
                                  num_cores=NC, num_subcores=NS)
    sc_heads = pl.kernel(
        _sc_heads_body,
        out_type=jax.ShapeDtypeStruct((B * D * HPAD,), jnp.float32),
        mesh=mesh,
        compiler_params=pltpu.CompilerParams(use_tc_tiling_on_sc=False,
                                             needs_layout_passes=False),
        scratch_types=[
            pltpu.VMEM((L,), jnp.float32),
            pltpu.VMEM((D,), jnp.int32),
            pltpu.VMEM((D,), jnp.int32),
            pltpu.VMEM((ZPW * HPAD,), jnp.float32),
            pltpu.SemaphoreType.DMA,
        ],
    )
    heads = sc_heads(schemas, logits, jnp.asarray(_ZHEAD)).reshape(B, D, HPAD)

    out_m = pl.pallas_call(
        _tc_mask_body,
        grid=(B,),
        in_specs=[pl.BlockSpec((1, 1, D), lambda i: (i, 0, 0))],
        out_specs=pl.BlockSpec((1, D, L), lambda i: (i, 0, 0)),
        out_shape=jax.ShapeDtypeStruct((B, D, L), jnp.int8),
    )(schemas.reshape(B, 1, D))

    out_l = pl.pallas_call(
        _tc_assemble_body,
        grid=(B,),
        in_specs=[pl.BlockSpec((1, D, HPAD), lambda i: (i, 0, 0))],
        out_specs=pl.BlockSpec((1, D, L), lambda i: (i, 0, 0)),
        out_shape=jax.ShapeDtypeStruct((B, D, L), jnp.float32),
    )(heads)
    return out_l, out_m.astype(jnp.bool_)


# speedup vs baseline: 56.7077x; 1.2139x over previous
import jax
import jax.numpy as jnp
import numpy as np
from jax import lax
from jax.experimental import pallas as pl
from jax.experimental.pallas import tpu as pltpu
from jax.experimental.pallas import tpu_sc as plsc

B, D, L = 16, 64, 4096
HEAD = 64
HPAD = 128
NC, NS = 2, 16
ZPW = D // 2
LANES = 16

_ZHEAD = np.zeros((NC * NS * ZPW * HPAD // (NC * NS),), np.float32)


def _sc_heads_body(schemas_hbm, logits_hbm, zf_hbm, heads_hbm,
                   logits_v, schemas_v, starts_v, hbuf, sem):
    cid = lax.axis_index("c")
    sid = lax.axis_index("s")
    wid = sid * NC + cid
    b = wid // 2
    d0 = (wid % 2) * ZPW

    pltpu.sync_copy(schemas_hbm.at[b], schemas_v)
    pltpu.sync_copy(logits_hbm.at[b], logits_v)
    pltpu.sync_copy(zf_hbm, hbuf)

    carry = jnp.int32(0)
    for ci in range(D // LANES):
        seg = schemas_v[pl.ds(ci * LANES, LANES)]
        inc = plsc.cumsum(seg)
        starts_v[pl.ds(ci * LANES, LANES)] = inc - seg + carry
        carry = carry + jnp.sum(seg)

    iota = lax.iota(jnp.int32, LANES)
    for z in range(ZPW):
        idxv = jnp.full((LANES,), d0 + z, jnp.int32)
        nd = plsc.load_gather(schemas_v, [idxv])
        sd = plsc.load_gather(starts_v, [idxv])
        for c4 in range(HEAD // LANES):
            j = iota + (c4 * LANES)
            m = j < nd
            gi = jnp.where(m, sd + j, 0)
            vals = plsc.load_gather(logits_v, [gi])
            hbuf[pl.ds(z * HPAD + c4 * LANES, LANES)] = jnp.where(m, vals, 0.0)
    pltpu.sync_copy(hbuf, heads_hbm.at[pl.ds(wid * (ZPW * HPAD), ZPW * HPAD)])


TB = 4


def _tc_mask_body(schemas_ref, outm_ref):
    col = lax.broadcasted_iota(jnp.int32, (TB, D, L), 2)
    n = schemas_ref[pl.ds(pl.program_id(0) * TB, TB), :]
    outm_ref[...] = (col < n[:, :, None]).astype(jnp.int8)


def _tc_assemble_body(heads_ref, outl_ref):
    outl_ref[:, :, pl.ds(0, HPAD)] = heads_ref[...]
    outl_ref[:, :, pl.ds(HPAD, L - HPAD)] = jnp.zeros((TB, D, L - HPAD),
                                                      jnp.float32)


def kernel(schemas, logits):
    schemas = schemas.astype(jnp.int32)
    logits = logits.astype(jnp.float32)

    mesh = plsc.VectorSubcoreMesh(core_axis_name="c", subcore_axis_name="s",
                                  num_cores=NC, num_subcores=NS)
    sc_heads = pl.kernel(
        _sc_heads_body,
        out_type=jax.ShapeDtypeStruct((B * D * HPAD,), jnp.float32),
        mesh=mesh,
        compiler_params=pltpu.CompilerParams(use_tc_tiling_on_sc=False,
                                             needs_layout_passes=False),
        scratch_types=[
            pltpu.VMEM((L,), jnp.float32),
            pltpu.VMEM((D,), jnp.int32),
            pltpu.VMEM((D,), jnp.int32),
            pltpu.VMEM((ZPW * HPAD,), jnp.float32),
            pltpu.SemaphoreType.DMA,
        ],
    )
    heads = sc_heads(schemas, logits, jnp.asarray(_ZHEAD)).reshape(B, D, HPAD)

    out_m = pl.pallas_call(
        _tc_mask_body,
        grid=(B // TB,),
        in_specs=[pl.BlockSpec((B, D), lambda i: (0, 0))],
        out_specs=pl.BlockSpec((TB, D, L), lambda i: (i, 0, 0)),
        out_shape=jax.ShapeDtypeStruct((B, D, L), jnp.int8),
    )(schemas)

    out_l = pl.pallas_call(
        _tc_assemble_body,
        grid=(B // TB,),
        in_specs=[pl.BlockSpec((TB, D, HPAD), lambda i: (i, 0, 0))],
        out_specs=pl.BlockSpec((TB, D, L), lambda i: (i, 0, 0)),
        out_shape=jax.ShapeDtypeStruct((B, D, L), jnp.float32),
    )(heads)
    return out_l, out_m.astype(jnp.bool_)


# speedup vs baseline: 57.2755x; 1.0100x over previous
import jax
import jax.numpy as jnp
import numpy as np
from jax import lax
from jax.experimental import pallas as pl
from jax.experimental.pallas import tpu as pltpu
from jax.experimental.pallas import tpu_sc as plsc

B, D, L = 16, 64, 4096
HEAD = 64
HPAD = 128
TAIL = L - HPAD
NC, NS = 2, 16
ZPW = D // 2
NBAND = ZPW // 8
LANES = 16
TB = 8

_ZTAIL = np.zeros((8, TAIL), np.float32)


def _sc_body(schemas_hbm, logits_hbm, ztail_hbm, out_hbm,
             logits_v, schemas_v, starts_v, ztail_v, htile_v, zsem, sem):
    cid = lax.axis_index("c")
    sid = lax.axis_index("s")
    wid = sid * NC + cid
    b = wid // 2
    d0 = (wid % 2) * ZPW

    pltpu.sync_copy(schemas_hbm.at[b], schemas_v)
    pltpu.sync_copy(logits_hbm.at[b], logits_v)
    pltpu.sync_copy(ztail_hbm, ztail_v)
    tails = [
        pltpu.async_copy(
            ztail_v, out_hbm.at[b, pl.ds(d0 + 8 * t, 8), pl.ds(HPAD, TAIL)], zsem)
        for t in range(NBAND)
    ]
    zv = jnp.zeros((LANES,), jnp.float32)
    for r in range(8):
        for c4 in range(HEAD // LANES):
            htile_v[r, pl.ds(HEAD + c4 * LANES, LANES)] = zv

    carry = jnp.int32(0)
    for ci in range(D // LANES):
        seg = schemas_v[pl.ds(ci * LANES, LANES)]
        inc = plsc.cumsum(seg)
        starts_v[pl.ds(ci * LANES, LANES)] = inc - seg + carry
        carry = carry + jnp.sum(seg)

    iota = lax.iota(jnp.int32, LANES)
    for t in range(NBAND):
        for r in range(8):
            idxv = jnp.full((LANES,), d0 + t * 8 + r, jnp.int32)
            nd = plsc.load_gather(schemas_v, [idxv])
            sd = plsc.load_gather(starts_v, [idxv])
            for c4 in range(HEAD // LANES):
                j = iota + (c4 * LANES)
                m = j < nd
                gi = jnp.where(m, sd + j, 0)
                vals = plsc.load_gather(logits_v, [gi])
                htile_v[r, pl.ds(c4 * LANES, LANES)] = jnp.where(m, vals, 0.0)
        pltpu.sync_copy(htile_v, out_hbm.at[b, pl.ds(d0 + 8 * t, 8), pl.ds(0, HPAD)])
    for cp in tails:
        cp.wait()


def _tc_mask_body(schemas_ref, outm_ref):
    col = lax.broadcasted_iota(jnp.int32, (TB, D, L), 2)
    n = schemas_ref[pl.ds(pl.program_id(0) * TB, TB), :]
    outm_ref[...] = (col < n[:, :, None]).astype(jnp.int8)


def kernel(schemas, logits):
    schemas = schemas.astype(jnp.int32)
    logits = logits.astype(jnp.float32)

    mesh = plsc.VectorSubcoreMesh(core_axis_name="c", subcore_axis_name="s",
                                  num_cores=NC, num_subcores=NS)
    sc_out = pl.kernel(
        _sc_body,
        out_type=jax.ShapeDtypeStruct((B, D, L), jnp.float32),
        mesh=mesh,
        compiler_params=pltpu.CompilerParams(use_tc_tiling_on_sc=True,
                                             needs_layout_passes=False),
        scratch_types=[
            pltpu.VMEM((L,), jnp.float32),
            pltpu.VMEM((D,), jnp.int32),
            pltpu.VMEM((D,), jnp.int32),
            pltpu.VMEM((8, TAIL), jnp.float32),
            pltpu.VMEM((8, HPAD), jnp.float32),
            pltpu.SemaphoreType.DMA,
            pltpu.SemaphoreType.DMA,
        ],
    )
    out_l = sc_out(schemas, logits, jnp.asarray(_ZTAIL))

    out_m = pl.pallas_call(
        _tc_mask_body,
        grid=(B // TB,),
        in_specs=[pl.BlockSpec((B, D), lambda i: (0, 0))],
        out_specs=pl.BlockSpec((TB, D, L), lambda i: (i, 0, 0)),
        out_shape=jax.ShapeDtypeStruct((B, D, L), jnp.int8),
    )(schemas)
    return out_l, out_m.astype(jnp.bool_)


# speedup vs baseline: 58.5070x; 1.0215x over previous
import jax
import jax.numpy as jnp
import numpy as np
from jax import lax
from jax.experimental import pallas as pl
from jax.experimental.pallas import tpu as pltpu
from jax.experimental.pallas import tpu_sc as plsc

B, D, L = 16, 64, 4096
HEAD = 64
HPAD = 128
TAIL = L - HPAD
NC, NS = 2, 16
ZPW = D // 2
NBAND = ZPW // 8
LANES = 16
TB = 8

_ZTAIL = np.zeros((8, TAIL), np.float32)


def _sc_body(schemas_hbm, logits_hbm, ztail_hbm, out_hbm,
             logits_v, schemas_v, starts_v, ztail_v, htile_v, zsem, sem):
    cid = lax.axis_index("c")
    sid = lax.axis_index("s")
    wid = sid * NC + cid
    b = wid // 2
    d0 = (wid % 2) * ZPW

    pltpu.sync_copy(schemas_hbm.at[b], schemas_v)
    pltpu.sync_copy(logits_hbm.at[b], logits_v)
    pltpu.sync_copy(ztail_hbm, ztail_v)
    tails = [
        pltpu.async_copy(
            ztail_v, out_hbm.at[b, pl.ds(d0 + 8 * t, 8), pl.ds(HPAD, TAIL)], zsem)
        for t in range(NBAND)
    ]
    zv = jnp.zeros((LANES,), jnp.float32)
    for buf in range(2):
        for r in range(8):
            for c4 in range(HEAD // LANES):
                htile_v[buf, r, pl.ds(HEAD + c4 * LANES, LANES)] = zv

    carry = jnp.int32(0)
    for ci in range(D // LANES):
        seg = schemas_v[pl.ds(ci * LANES, LANES)]
        inc = plsc.cumsum(seg)
        starts_v[pl.ds(ci * LANES, LANES)] = inc - seg + carry
        carry = carry + jnp.sum(seg)

    iota = lax.iota(jnp.int32, LANES)
    heads = []
    for t in range(NBAND):
        buf = t % 2
        if t >= 2:
            heads[t - 2].wait()
        for r in range(8):
            idxv = jnp.full((LANES,), d0 + t * 8 + r, jnp.int32)
            nd = plsc.load_gather(schemas_v, [idxv])
            sd = plsc.load_gather(starts_v, [idxv])
            for c4 in range(HEAD // LANES):
                j = iota + (c4 * LANES)
                m = j < nd
                gi = jnp.where(m, sd + j, 0)
                vals = plsc.load_gather(logits_v, [gi])
                htile_v[buf, r, pl.ds(c4 * LANES, LANES)] = jnp.where(m, vals, 0.0)
        heads.append(pltpu.async_copy(
            htile_v.at[buf], out_hbm.at[b, pl.ds(d0 + 8 * t, 8), pl.ds(0, HPAD)],
            sem))
    for cp in heads[-2:]:
        cp.wait()
    for cp in tails:
        cp.wait()


def _tc_mask_body(schemas_ref, outm_ref):
    col = lax.broadcasted_iota(jnp.int32, (B, D, HPAD), 2)
    n = schemas_ref[...]
    outm_ref[...] = (col < n[:, :, None]).astype(jnp.int8)


def kernel(schemas, logits):
    schemas = schemas.astype(jnp.int32)
    logits = logits.astype(jnp.float32)

    mesh = plsc.VectorSubcoreMesh(core_axis_name="c", subcore_axis_name="s",
                                  num_cores=NC, num_subcores=NS)
    sc_out = pl.kernel(
        _sc_body,
        out_type=jax.ShapeDtypeStruct((B, D, L), jnp.float32),
        mesh=mesh,
        compiler_params=pltpu.CompilerParams(use_tc_tiling_on_sc=True,
                                             needs_layout_passes=False),
        scratch_types=[
            pltpu.VMEM((L,), jnp.float32),
            pltpu.VMEM((D,), jnp.int32),
            pltpu.VMEM((D,), jnp.int32),
            pltpu.VMEM((8, TAIL), jnp.float32),
            pltpu.VMEM((2, 8, HPAD), jnp.float32),
            pltpu.SemaphoreType.DMA,
            pltpu.SemaphoreType.DMA,
        ],
    )
    out_l = sc_out(schemas, logits, jnp.asarray(_ZTAIL))

    mask_head = pl.pallas_call(
        _tc_mask_body,
        out_shape=jax.ShapeDtypeStruct((B, D, HPAD), jnp.int8),
    )(schemas)
    out_m = jnp.concatenate(
        [mask_head.astype(jnp.bool_),
         jnp.zeros((B, D, L - HPAD), jnp.bool_)], axis=-1)
    return out_l, out_m
